# baseline (speedup 1.0000x reference)
import functools

import jax
import jax.numpy as jnp
from jax import lax
from jax.experimental import pallas as pl
from jax.experimental.pallas import tpu as pltpu
from jax.experimental.pallas import tpu_sc as plsc

F32 = jnp.float32

_N = 10000
_E = 320000
_D = 128
_H = 128
_T = 1000
_NMOL = 256
_NB = 1000
_BE = 3200
_NWORK = 32
_EPW = _E // _NWORK
_CS = 80
_NCH = _EPW // _CS
_EPT = _E // 16
_NCH4 = _EPT // _CS


def _stage1a(pos0_ref, pos1_ref, seg_ref, t_ref, tbl_ref, stats_ref, mvec_ref):
    i = pl.program_id(0)
    t = t_ref[...]
    oh_t = (t == lax.broadcasted_iota(jnp.int32, (_NB, _T), 1)).astype(F32)
    mm = jnp.dot(oh_t, tbl_ref[...], preferred_element_type=F32)
    m0 = mm[:, 0:1]
    m1 = mm[:, 1:2]
    mvec_ref[...] = jnp.concatenate([mm, jnp.zeros((_NB, 1), F32)], axis=1)
    q = m0 * pos0_ref[...] + m1 * pos1_ref[...]
    seg = seg_ref[...]
    S = (seg == lax.broadcasted_iota(jnp.int32, (_NB, _NMOL), 1)).astype(F32)
    vals = jnp.concatenate(
        [pos1_ref[...], q, m1, jnp.ones((_NB, 1), F32)], axis=1)
    part = lax.dot_general(S, vals, (((0,), (0,)), ((), ())),
                           preferred_element_type=F32)

    @pl.when(i == 0)
    def _():
        stats_ref[...] = jnp.zeros_like(stats_ref)

    stats_ref[...] += part


def _stage1b(pos0_ref, pos1_ref, h_ref, seg_ref, mvec_ref, stats_ref,
             wa_ref, wb_ref, a_ref, b_ref, xt4_ref, label_ref):
    stats = stats_ref[...]
    cnt = jnp.maximum(stats[:, 7:8], 1.0)
    mean1 = stats[:, 0:3] / cnt
    meanxt = (stats[:, 3:6] - stats[:, 6:7] * mean1) / cnt
    meanmat = jnp.concatenate([mean1, meanxt], axis=1)
    seg = seg_ref[...]
    S = (seg == lax.broadcasted_iota(jnp.int32, (_NB, _NMOL), 1)).astype(F32)
    pm = jnp.dot(S, meanmat, preferred_element_type=F32)
    mv = mvec_ref[...]
    m0 = mv[:, 0:1]
    m1 = mv[:, 1:2]
    std = mv[:, 2:3]
    pos0 = pos0_ref[...]
    q = m0 * pos0 + m1 * pos1_ref[...]
    xt = q - m1 * pm[:, 0:3] - pm[:, 3:6]
    lbl = (xt - pos0) / std
    zpad = jnp.zeros((_NB, 1), F32)
    xt4_ref[...] = jnp.concatenate([xt, zpad], axis=1)
    label_ref[...] = jnp.concatenate([lbl, zpad], axis=1)
    hv = h_ref[...]
    a_ref[...] = jnp.dot(hv, wa_ref[...], preferred_element_type=F32)
    b_ref[...] = jnp.dot(hv, wb_ref[...], preferred_element_type=F32)


def _stage3(ga_ref, gb_ref, relr_ref, wl_ref, bm1_ref, wm2_ref, bm2_ref,
            wx_ref, o1_ref, ox_ref):
    pre = ga_ref[...] + gb_ref[...]
    rel = relr_ref[...][:, 0:3]
    d2 = jnp.sum(rel * rel, axis=1, keepdims=True)
    z1 = pre + d2 * wl_ref[...] + bm1_ref[...]
    m1 = z1 * jax.nn.sigmoid(z1)
    z2 = jnp.dot(m1, wm2_ref[...], preferred_element_type=F32) + bm2_ref[...]
    m = z2 * jax.nn.sigmoid(z2)
    cw = jnp.tanh(jnp.dot(m, wx_ref[...], preferred_element_type=F32))
    o1_ref[...] = m
    pad = jnp.zeros((_BE, 13), F32)
    ox_ref[...] = jnp.concatenate([rel * cw, pad], axis=1)


def _stage5(agm0_ref, agm1_ref, agx0_ref, agx1_ref, h_ref, label_ref,
            wht_ref, whb_ref, bh_ref, out_ref):
    aggm = agm0_ref[...] + agm1_ref[...]
    aggx = (agx0_ref[...] + agx1_ref[...])[:, 0:3]
    hv = h_ref[...]
    z = (jnp.dot(hv, wht_ref[...], preferred_element_type=F32)
         + jnp.dot(aggm, whb_ref[...], preferred_element_type=F32)
         + bh_ref[...])
    ho = hv + z * jax.nn.sigmoid(z)
    diff = aggx - label_ref[...][:, 0:3]
    loss = (jnp.sum(diff * diff) / (_N * 3)
            + 1e-4 * jnp.sum(ho * ho) / (_N * _D))
    out_ref[...] = loss.reshape(1, 1)


def _sc_gather(dst_hbm, src_hbm, a_hbm, b_hbm, xt4_hbm,
               ga_hbm, gb_hbm, relr_hbm,
               xt4v, idxd, idxs, bufa, bufb, bufr, sema, semb):
    c = lax.axis_index("c")
    s = lax.axis_index("s")
    wid = s * 2 + c
    base = wid * _EPW
    pltpu.sync_copy(xt4_hbm, xt4v)
    col = jnp.minimum(lax.iota(jnp.int32, 16), 3)

    def body(k, carry):
        off = base + k * _CS
        pltpu.sync_copy(dst_hbm.at[pl.ds(off, _CS)], idxd)
        pltpu.sync_copy(src_hbm.at[pl.ds(off, _CS)], idxs)
        da = pltpu.async_copy(a_hbm.at[idxd], bufa, sema)
        db = pltpu.async_copy(b_hbm.at[idxs], bufb, semb)

        def ebody(e, cc):
            ev = jnp.zeros((16,), jnp.int32) + e
            dn = plsc.load_gather(idxd, [ev])
            sn = plsc.load_gather(idxs, [ev])
            xd = plsc.load_gather(xt4v, [dn, col])
            xs = plsc.load_gather(xt4v, [sn, col])
            bufr[e, pl.ds(0, 16)] = xd - xs
            return cc

        lax.fori_loop(0, _CS, ebody, 0)
        da.wait()
        db.wait()
        pltpu.sync_copy(bufa, ga_hbm.at[pl.ds(off, _CS)])
        pltpu.sync_copy(bufb, gb_hbm.at[pl.ds(off, _CS)])
        pltpu.sync_copy(bufr, relr_hbm.at[pl.ds(off, _CS)])
        return carry

    lax.fori_loop(0, _NCH, body, 0)


def _sc_scatter(o1_hbm, ox_hbm, dst_hbm, zm_hbm, zx_hbm,
                agm0_hbm, agm1_hbm, agx0_hbm, agx1_hbm,
                accm, accx, idxv, bufm, bufx):
    c = lax.axis_index("c")
    s = lax.axis_index("s")
    rpw = _N // 16
    pltpu.sync_copy(zm_hbm.at[pl.ds(s * rpw, rpw)],
                    accm.at[pl.ds(s * rpw, rpw)])
    pltpu.sync_copy(zx_hbm.at[pl.ds(s * rpw, rpw)],
                    accx.at[pl.ds(s * rpw, rpw)])
    plsc.subcore_barrier()
    base = (c * 16 + s) * _EPW

    def body(k, carry):
        off = base + k * _CS
        pltpu.sync_copy(dst_hbm.at[pl.ds(off, _CS)], idxv)
        pltpu.sync_copy(o1_hbm.at[pl.ds(off, _CS)], bufm)
        pltpu.sync_copy(ox_hbm.at[pl.ds(off, _CS)], bufx)
        pltpu.sync_copy(bufm, accm.at[idxv], add=True)
        pltpu.sync_copy(bufx, accx.at[idxv], add=True)
        return carry

    lax.fori_loop(0, _NCH, body, 0)
    plsc.subcore_barrier()

    @pl.when(c == 0)
    def _():
        pltpu.sync_copy(accm.at[pl.ds(s * rpw, rpw)],
                        agm0_hbm.at[pl.ds(s * rpw, rpw)])
        pltpu.sync_copy(accx.at[pl.ds(s * rpw, rpw)],
                        agx0_hbm.at[pl.ds(s * rpw, rpw)])

    @pl.when(c == 1)
    def _():
        pltpu.sync_copy(accm.at[pl.ds(s * rpw, rpw)],
                        agm1_hbm.at[pl.ds(s * rpw, rpw)])
        pltpu.sync_copy(accx.at[pl.ds(s * rpw, rpw)],
                        agx1_hbm.at[pl.ds(s * rpw, rpw)])


def kernel(pos0, pos1, h, Wm1, bm1, Wm2, bm2, Wx, Wh, bh, edge_index,
           seg_ids, t_step):
    betas = jnp.linspace(1e-5, 1e-2, _T, dtype=F32)
    var_fwd = jnp.cumsum(betas)
    var_bwd = jnp.flip(jnp.cumsum(jnp.flip(betas)))
    std_fwd = jnp.sqrt(var_fwd)
    denom = var_fwd + var_bwd
    tbl = jnp.stack([var_bwd / denom, var_fwd / denom, std_fwd], axis=1)

    seg2d = seg_ids.reshape(_N, 1)
    t2d = t_step.reshape(_N, 1)
    srci = edge_index[0]
    dsti = edge_index[1]
    nblk = _N // _NB

    stats, mvec = pl.pallas_call(
        _stage1a,
        grid=(nblk,),
        in_specs=[
            pl.BlockSpec((_NB, 3), lambda i: (i, 0)),
            pl.BlockSpec((_NB, 3), lambda i: (i, 0)),
            pl.BlockSpec((_NB, 1), lambda i: (i, 0)),
            pl.BlockSpec((_NB, 1), lambda i: (i, 0)),
            pl.BlockSpec((_T, 3), lambda i: (0, 0)),
        ],
        out_specs=[
            pl.BlockSpec((_NMOL, 8), lambda i: (0, 0)),
            pl.BlockSpec((_NB, 4), lambda i: (i, 0)),
        ],
        out_shape=[
            jax.ShapeDtypeStruct((_NMOL, 8), F32),
            jax.ShapeDtypeStruct((_N, 4), F32),
        ],
    )(pos0, pos1, seg2d, t2d, tbl)

    A, B, xt4, label4 = pl.pallas_call(
        _stage1b,
        grid=(nblk,),
        in_specs=[
            pl.BlockSpec((_NB, 3), lambda i: (i, 0)),
            pl.BlockSpec((_NB, 3), lambda i: (i, 0)),
            pl.BlockSpec((_NB, _D), lambda i: (i, 0)),
            pl.BlockSpec((_NB, 1), lambda i: (i, 0)),
            pl.BlockSpec((_NB, 4), lambda i: (i, 0)),
            pl.BlockSpec((_NMOL, 8), lambda i: (0, 0)),
            pl.BlockSpec((_D, _H), lambda i: (0, 0)),
            pl.BlockSpec((_D, _H), lambda i: (0, 0)),
        ],
        out_specs=[
            pl.BlockSpec((_NB, _H), lambda i: (i, 0)),
            pl.BlockSpec((_NB, _H), lambda i: (i, 0)),
            pl.BlockSpec((_NB, 4), lambda i: (i, 0)),
            pl.BlockSpec((_NB, 4), lambda i: (i, 0)),
        ],
        out_shape=[
            jax.ShapeDtypeStruct((_N, _H), F32),
            jax.ShapeDtypeStruct((_N, _H), F32),
            jax.ShapeDtypeStruct((_N, 4), F32),
            jax.ShapeDtypeStruct((_N, 4), F32),
        ],
    )(pos0, pos1, h, seg2d, mvec, stats, Wm1[:_D], Wm1[_D:2 * _D])

    mesh = plsc.VectorSubcoreMesh(core_axis_name="c", subcore_axis_name="s")
    gather = functools.partial(
        pl.kernel,
        out_type=[
            jax.ShapeDtypeStruct((_E, _H), F32),
            jax.ShapeDtypeStruct((_E, _H), F32),
            jax.ShapeDtypeStruct((_E, 16), F32),
        ],
        mesh=mesh,
        compiler_params=pltpu.CompilerParams(use_tc_tiling_on_sc=False, needs_layout_passes=False),
        scratch_types=[
            pltpu.VMEM((_N, 4), F32),
            pltpu.VMEM((_CS,), jnp.int32),
            pltpu.VMEM((_CS,), jnp.int32),
            pltpu.VMEM((_CS, _H), F32),
            pltpu.VMEM((_CS, _H), F32),
            pltpu.VMEM((_CS, 16), F32),
            pltpu.SemaphoreType.DMA,
            pltpu.SemaphoreType.DMA,
        ],
    )(_sc_gather)
    GA, GB, RELR = gather(dsti, srci, A, B, xt4)

    geb = _E // _BE
    O1, OX = pl.pallas_call(
        _stage3,
        grid=(geb,),
        in_specs=[
            pl.BlockSpec((_BE, _H), lambda i: (i, 0)),
            pl.BlockSpec((_BE, _H), lambda i: (i, 0)),
            pl.BlockSpec((_BE, 16), lambda i: (i, 0)),
            pl.BlockSpec((1, _H), lambda i: (0, 0)),
            pl.BlockSpec((1, _H), lambda i: (0, 0)),
            pl.BlockSpec((_H, _H), lambda i: (0, 0)),
            pl.BlockSpec((1, _H), lambda i: (0, 0)),
            pl.BlockSpec((_H, 1), lambda i: (0, 0)),
        ],
        out_specs=[
            pl.BlockSpec((_BE, _H), lambda i: (i, 0)),
            pl.BlockSpec((_BE, 16), lambda i: (i, 0)),
        ],
        out_shape=[
            jax.ShapeDtypeStruct((_E, _H), F32),
            jax.ShapeDtypeStruct((_E, 16), F32),
        ],
    )(GA, GB, RELR, Wm1[2 * _D].reshape(1, _H), bm1.reshape(1, _H), Wm2,
      bm2.reshape(1, _H), Wx)

    mesh2 = plsc.VectorSubcoreMesh(core_axis_name="c", subcore_axis_name="s")
    scatter = functools.partial(
        pl.kernel,
        out_type=[
            jax.ShapeDtypeStruct((_N, _H), F32),
            jax.ShapeDtypeStruct((_N, _H), F32),
            jax.ShapeDtypeStruct((_N, 16), F32),
            jax.ShapeDtypeStruct((_N, 16), F32),
        ],
        mesh=mesh2,
        compiler_params=pltpu.CompilerParams(use_tc_tiling_on_sc=False, needs_layout_passes=False),
        scratch_types=[
            pltpu.VMEM_SHARED((_N, _H), F32),
            pltpu.VMEM_SHARED((_N, 16), F32),
            pltpu.VMEM((_CS,), jnp.int32),
            pltpu.VMEM((_CS, _H), F32),
            pltpu.VMEM((_CS, 16), F32),
        ],
    )(_sc_scatter)
    AGM0, AGM1, AGX0, AGX1 = scatter(O1, OX, dsti, jnp.zeros((_N, _H), F32),
                                     jnp.zeros((_N, 16), F32))

    loss2d = pl.pallas_call(
        _stage5,
        grid=(1,),
        in_specs=[
            pl.BlockSpec((_N, _H), lambda i: (0, 0)),
            pl.BlockSpec((_N, _H), lambda i: (0, 0)),
            pl.BlockSpec((_N, 16), lambda i: (0, 0)),
            pl.BlockSpec((_N, 16), lambda i: (0, 0)),
            pl.BlockSpec((_N, _D), lambda i: (0, 0)),
            pl.BlockSpec((_N, 4), lambda i: (0, 0)),
            pl.BlockSpec((_D, _D), lambda i: (0, 0)),
            pl.BlockSpec((_D, _D), lambda i: (0, 0)),
            pl.BlockSpec((1, _D), lambda i: (0, 0)),
        ],
        out_specs=pl.BlockSpec((1, 1), lambda i: (0, 0)),
        out_shape=jax.ShapeDtypeStruct((1, 1), F32),
    )(AGM0, AGM1, AGX0, AGX1, h, label4, Wh[:_D], Wh[_D:], bh.reshape(1, _D))

    return loss2d[0, 0]


# speedup vs baseline: 1.1099x; 1.1099x over previous
import functools

import jax
import jax.numpy as jnp
from jax import lax
from jax.experimental import pallas as pl
from jax.experimental.pallas import tpu as pltpu
from jax.experimental.pallas import tpu_sc as plsc

F32 = jnp.float32

_N = 10000
_E = 320000
_D = 128
_H = 128
_T = 1000
_NMOL = 256
_NB = 1000
_BE = 3200
_NWORK = 32
_EPW = _E // _NWORK
_CS = 80
_NCH = _EPW // _CS
_EPT = _E // 16
_NCH4 = _EPT // _CS


def _stage1a(pos0_ref, pos1_ref, seg_ref, t_ref, tbl_ref, stats_ref, mvec_ref):
    i = pl.program_id(0)
    t = t_ref[...]
    oh_t = (t == lax.broadcasted_iota(jnp.int32, (_NB, _T), 1)).astype(F32)
    mm = jnp.dot(oh_t, tbl_ref[...], preferred_element_type=F32)
    m0 = mm[:, 0:1]
    m1 = mm[:, 1:2]
    mvec_ref[...] = jnp.concatenate([mm, jnp.zeros((_NB, 1), F32)], axis=1)
    q = m0 * pos0_ref[...] + m1 * pos1_ref[...]
    seg = seg_ref[...]
    S = (seg == lax.broadcasted_iota(jnp.int32, (_NB, _NMOL), 1)).astype(F32)
    vals = jnp.concatenate(
        [pos1_ref[...], q, m1, jnp.ones((_NB, 1), F32)], axis=1)
    part = lax.dot_general(S, vals, (((0,), (0,)), ((), ())),
                           preferred_element_type=F32)

    @pl.when(i == 0)
    def _():
        stats_ref[...] = jnp.zeros_like(stats_ref)

    stats_ref[...] += part


def _stage1b(pos0_ref, pos1_ref, h_ref, seg_ref, mvec_ref, stats_ref,
             wa_ref, wb_ref, a_ref, b_ref, xt4_ref, label_ref):
    stats = stats_ref[...]
    cnt = jnp.maximum(stats[:, 7:8], 1.0)
    mean1 = stats[:, 0:3] / cnt
    meanxt = (stats[:, 3:6] - stats[:, 6:7] * mean1) / cnt
    meanmat = jnp.concatenate([mean1, meanxt], axis=1)
    seg = seg_ref[...]
    S = (seg == lax.broadcasted_iota(jnp.int32, (_NB, _NMOL), 1)).astype(F32)
    pm = jnp.dot(S, meanmat, preferred_element_type=F32)
    mv = mvec_ref[...]
    m0 = mv[:, 0:1]
    m1 = mv[:, 1:2]
    std = mv[:, 2:3]
    pos0 = pos0_ref[...]
    q = m0 * pos0 + m1 * pos1_ref[...]
    xt = q - m1 * pm[:, 0:3] - pm[:, 3:6]
    lbl = (xt - pos0) / std
    zpad = jnp.zeros((_NB, 1), F32)
    xt4_ref[...] = jnp.concatenate([xt, zpad], axis=1)
    label_ref[...] = jnp.concatenate([lbl, zpad], axis=1)
    hv = h_ref[...]
    a_ref[...] = jnp.dot(hv, wa_ref[...], preferred_element_type=F32)
    b_ref[...] = jnp.dot(hv, wb_ref[...], preferred_element_type=F32)


def _stage3(ga_ref, gb_ref, relr_ref, wl_ref, bm1_ref, wm2_ref, bm2_ref,
            wx_ref, o1_ref, ox_ref):
    pre = ga_ref[...] + gb_ref[...]
    relt = relr_ref[...]
    sel3 = (lax.broadcasted_iota(jnp.int32, (8, 1), 0) < 3).astype(F32)
    d2 = lax.dot_general(relt * relt, sel3, (((0,), (0,)), ((), ())),
                         preferred_element_type=F32)
    z1 = pre + d2 * wl_ref[...] + bm1_ref[...]
    m1 = z1 * jax.nn.sigmoid(z1)
    z2 = jnp.dot(m1, wm2_ref[...], preferred_element_type=F32) + bm2_ref[...]
    m = z2 * jax.nn.sigmoid(z2)
    cwt = jnp.tanh(lax.dot_general(wx_ref[...], m, (((1,), (1,)), ((), ())),
                                   preferred_element_type=F32))
    o1_ref[...] = m
    ox_ref[...] = relt * cwt


def _stage5(agm0_ref, agm1_ref, agx0_ref, agx1_ref, h_ref, label_ref,
            wht_ref, whb_ref, bh_ref, out_ref):
    aggm = agm0_ref[...] + agm1_ref[...]
    aggx = (agx0_ref[...] + agx1_ref[...])[:, 0:3]
    hv = h_ref[...]
    z = (jnp.dot(hv, wht_ref[...], preferred_element_type=F32)
         + jnp.dot(aggm, whb_ref[...], preferred_element_type=F32)
         + bh_ref[...])
    ho = hv + z * jax.nn.sigmoid(z)
    diff = aggx - label_ref[...][:, 0:3]
    loss = (jnp.sum(diff * diff) / (_N * 3)
            + 1e-4 * jnp.sum(ho * ho) / (_N * _D))
    out_ref[...] = loss.reshape(1, 1)


def _sc_gather(dst_hbm, src_hbm, a_hbm, b_hbm, xt4_hbm,
               ga_hbm, gb_hbm, relr_hbm,
               xt4v, idxd, idxs, bufa, bufb, bufr, sema, semb):
    c = lax.axis_index("c")
    s = lax.axis_index("s")
    wid = s * 2 + c
    base = wid * _EPW
    pltpu.sync_copy(xt4_hbm, xt4v)
    lane = lax.iota(jnp.int32, 16)
    col = jnp.minimum(lane, 3)
    mask4 = lane < 4
    z16 = jnp.zeros((16,), F32)
    for r in range(8):
        for j in range(_CS // 16):
            bufr[r, pl.ds(j * 16, 16)] = z16

    def body(k, carry):
        off = base + k * _CS
        pltpu.sync_copy(dst_hbm.at[pl.ds(off, _CS)], idxd)
        pltpu.sync_copy(src_hbm.at[pl.ds(off, _CS)], idxs)
        da = pltpu.async_copy(a_hbm.at[idxd], bufa, sema)
        db = pltpu.async_copy(b_hbm.at[idxs], bufb, semb)

        def ebody(r, cc):
            for g in range(8):
                e = r * 8 + g
                ev = jnp.zeros((16,), jnp.int32) + e
                dn = plsc.load_gather(idxd, [ev])
                sn = plsc.load_gather(idxs, [ev])
                xd = plsc.load_gather(xt4v, [dn, col])
                xs = plsc.load_gather(xt4v, [sn, col])
                plsc.store_scatter(bufr, [col, ev], xd - xs, mask=mask4)
            return cc

        lax.fori_loop(0, _CS // 8, ebody, 0)
        da.wait()
        db.wait()
        pltpu.sync_copy(bufa, ga_hbm.at[pl.ds(off, _CS)])
        pltpu.sync_copy(bufb, gb_hbm.at[pl.ds(off, _CS)])
        pltpu.sync_copy(bufr, relr_hbm.at[:, pl.ds(off, _CS)])
        return carry

    lax.fori_loop(0, _NCH, body, 0)


def _sc_scatter(o1_hbm, ox_hbm, dst_hbm, zm_hbm, zx_hbm,
                agm0_hbm, agm1_hbm, agx0_hbm, agx1_hbm,
                accm, accx, idxv, bufm, bufxt, bufx):
    c = lax.axis_index("c")
    s = lax.axis_index("s")
    lane = lax.iota(jnp.int32, 16)
    col = jnp.minimum(lane, 3)
    rpw = _N // 16
    pltpu.sync_copy(zm_hbm.at[pl.ds(s * rpw, rpw)],
                    accm.at[pl.ds(s * rpw, rpw)])
    pltpu.sync_copy(zx_hbm.at[pl.ds(s * rpw, rpw)],
                    accx.at[pl.ds(s * rpw, rpw)])
    plsc.subcore_barrier()
    base = (c * 16 + s) * _EPW

    def body(k, carry):
        off = base + k * _CS
        pltpu.sync_copy(dst_hbm.at[pl.ds(off, _CS)], idxv)
        pltpu.sync_copy(o1_hbm.at[pl.ds(off, _CS)], bufm)
        pltpu.sync_copy(ox_hbm.at[:, pl.ds(off, _CS)], bufxt)

        def ebody(r, cc):
            for g in range(8):
                e = r * 8 + g
                ev = jnp.zeros((16,), jnp.int32) + e
                bufx[e, pl.ds(0, 16)] = plsc.load_gather(bufxt, [col, ev])
            return cc

        lax.fori_loop(0, _CS // 8, ebody, 0)
        pltpu.sync_copy(bufm, accm.at[idxv], add=True)
        pltpu.sync_copy(bufx, accx.at[idxv], add=True)
        return carry

    lax.fori_loop(0, _NCH, body, 0)
    plsc.subcore_barrier()

    @pl.when(c == 0)
    def _():
        pltpu.sync_copy(accm.at[pl.ds(s * rpw, rpw)],
                        agm0_hbm.at[pl.ds(s * rpw, rpw)])
        pltpu.sync_copy(accx.at[pl.ds(s * rpw, rpw)],
                        agx0_hbm.at[pl.ds(s * rpw, rpw)])

    @pl.when(c == 1)
    def _():
        pltpu.sync_copy(accm.at[pl.ds(s * rpw, rpw)],
                        agm1_hbm.at[pl.ds(s * rpw, rpw)])
        pltpu.sync_copy(accx.at[pl.ds(s * rpw, rpw)],
                        agx1_hbm.at[pl.ds(s * rpw, rpw)])


def kernel(pos0, pos1, h, Wm1, bm1, Wm2, bm2, Wx, Wh, bh, edge_index,
           seg_ids, t_step):
    betas = jnp.linspace(1e-5, 1e-2, _T, dtype=F32)
    var_fwd = jnp.cumsum(betas)
    var_bwd = jnp.flip(jnp.cumsum(jnp.flip(betas)))
    std_fwd = jnp.sqrt(var_fwd)
    denom = var_fwd + var_bwd
    tbl = jnp.stack([var_bwd / denom, var_fwd / denom, std_fwd], axis=1)

    seg2d = seg_ids.reshape(_N, 1)
    t2d = t_step.reshape(_N, 1)
    srci = edge_index[0]
    dsti = edge_index[1]
    nblk = _N // _NB

    stats, mvec = pl.pallas_call(
        _stage1a,
        grid=(nblk,),
        in_specs=[
            pl.BlockSpec((_NB, 3), lambda i: (i, 0)),
            pl.BlockSpec((_NB, 3), lambda i: (i, 0)),
            pl.BlockSpec((_NB, 1), lambda i: (i, 0)),
            pl.BlockSpec((_NB, 1), lambda i: (i, 0)),
            pl.BlockSpec((_T, 3), lambda i: (0, 0)),
        ],
        out_specs=[
            pl.BlockSpec((_NMOL, 8), lambda i: (0, 0)),
            pl.BlockSpec((_NB, 4), lambda i: (i, 0)),
        ],
        out_shape=[
            jax.ShapeDtypeStruct((_NMOL, 8), F32),
            jax.ShapeDtypeStruct((_N, 4), F32),
        ],
    )(pos0, pos1, seg2d, t2d, tbl)

    A, B, xt4, label4 = pl.pallas_call(
        _stage1b,
        grid=(nblk,),
        in_specs=[
            pl.BlockSpec((_NB, 3), lambda i: (i, 0)),
            pl.BlockSpec((_NB, 3), lambda i: (i, 0)),
            pl.BlockSpec((_NB, _D), lambda i: (i, 0)),
            pl.BlockSpec((_NB, 1), lambda i: (i, 0)),
            pl.BlockSpec((_NB, 4), lambda i: (i, 0)),
            pl.BlockSpec((_NMOL, 8), lambda i: (0, 0)),
            pl.BlockSpec((_D, _H), lambda i: (0, 0)),
            pl.BlockSpec((_D, _H), lambda i: (0, 0)),
        ],
        out_specs=[
            pl.BlockSpec((_NB, _H), lambda i: (i, 0)),
            pl.BlockSpec((_NB, _H), lambda i: (i, 0)),
            pl.BlockSpec((_NB, 4), lambda i: (i, 0)),
            pl.BlockSpec((_NB, 4), lambda i: (i, 0)),
        ],
        out_shape=[
            jax.ShapeDtypeStruct((_N, _H), F32),
            jax.ShapeDtypeStruct((_N, _H), F32),
            jax.ShapeDtypeStruct((_N, 4), F32),
            jax.ShapeDtypeStruct((_N, 4), F32),
        ],
    )(pos0, pos1, h, seg2d, mvec, stats, Wm1[:_D], Wm1[_D:2 * _D])

    mesh = plsc.VectorSubcoreMesh(core_axis_name="c", subcore_axis_name="s")
    gather = functools.partial(
        pl.kernel,
        out_type=[
            jax.ShapeDtypeStruct((_E, _H), F32),
            jax.ShapeDtypeStruct((_E, _H), F32),
            jax.ShapeDtypeStruct((8, _E), F32),
        ],
        mesh=mesh,
        compiler_params=pltpu.CompilerParams(use_tc_tiling_on_sc=False, needs_layout_passes=False),
        scratch_types=[
            pltpu.VMEM((_N, 4), F32),
            pltpu.VMEM((_CS,), jnp.int32),
            pltpu.VMEM((_CS,), jnp.int32),
            pltpu.VMEM((_CS, _H), F32),
            pltpu.VMEM((_CS, _H), F32),
            pltpu.VMEM((8, _CS), F32),
            pltpu.SemaphoreType.DMA,
            pltpu.SemaphoreType.DMA,
        ],
    )(_sc_gather)
    GA, GB, RELR = gather(dsti, srci, A, B, xt4)

    geb = _E // _BE
    O1, OX = pl.pallas_call(
        _stage3,
        grid=(geb,),
        in_specs=[
            pl.BlockSpec((_BE, _H), lambda i: (i, 0)),
            pl.BlockSpec((_BE, _H), lambda i: (i, 0)),
            pl.BlockSpec((8, _BE), lambda i: (0, i)),
            pl.BlockSpec((1, _H), lambda i: (0, 0)),
            pl.BlockSpec((1, _H), lambda i: (0, 0)),
            pl.BlockSpec((_H, _H), lambda i: (0, 0)),
            pl.BlockSpec((1, _H), lambda i: (0, 0)),
            pl.BlockSpec((1, _H), lambda i: (0, 0)),
        ],
        out_specs=[
            pl.BlockSpec((_BE, _H), lambda i: (i, 0)),
            pl.BlockSpec((8, _BE), lambda i: (0, i)),
        ],
        out_shape=[
            jax.ShapeDtypeStruct((_E, _H), F32),
            jax.ShapeDtypeStruct((8, _E), F32),
        ],
    )(GA, GB, RELR, Wm1[2 * _D].reshape(1, _H), bm1.reshape(1, _H), Wm2,
      bm2.reshape(1, _H), Wx.reshape(1, _H))

    mesh2 = plsc.VectorSubcoreMesh(core_axis_name="c", subcore_axis_name="s")
    scatter = functools.partial(
        pl.kernel,
        out_type=[
            jax.ShapeDtypeStruct((_N, _H), F32),
            jax.ShapeDtypeStruct((_N, _H), F32),
            jax.ShapeDtypeStruct((_N, 16), F32),
            jax.ShapeDtypeStruct((_N, 16), F32),
        ],
        mesh=mesh2,
        compiler_params=pltpu.CompilerParams(use_tc_tiling_on_sc=False, needs_layout_passes=False),
        scratch_types=[
            pltpu.VMEM_SHARED((_N, _H), F32),
            pltpu.VMEM_SHARED((_N, 16), F32),
            pltpu.VMEM((_CS,), jnp.int32),
            pltpu.VMEM((_CS, _H), F32),
            pltpu.VMEM((8, _CS), F32),
            pltpu.VMEM((_CS, 16), F32),
        ],
    )(_sc_scatter)
    AGM0, AGM1, AGX0, AGX1 = scatter(O1, OX, dsti, jnp.zeros((_N, _H), F32),
                                     jnp.zeros((_N, 16), F32))

    loss2d = pl.pallas_call(
        _stage5,
        grid=(1,),
        in_specs=[
            pl.BlockSpec((_N, _H), lambda i: (0, 0)),
            pl.BlockSpec((_N, _H), lambda i: (0, 0)),
            pl.BlockSpec((_N, 16), lambda i: (0, 0)),
            pl.BlockSpec((_N, 16), lambda i: (0, 0)),
            pl.BlockSpec((_N, _D), lambda i: (0, 0)),
            pl.BlockSpec((_N, 4), lambda i: (0, 0)),
            pl.BlockSpec((_D, _D), lambda i: (0, 0)),
            pl.BlockSpec((_D, _D), lambda i: (0, 0)),
            pl.BlockSpec((1, _D), lambda i: (0, 0)),
        ],
        out_specs=pl.BlockSpec((1, 1), lambda i: (0, 0)),
        out_shape=jax.ShapeDtypeStruct((1, 1), F32),
    )(AGM0, AGM1, AGX0, AGX1, h, label4, Wh[:_D], Wh[_D:], bh.reshape(1, _D))

    return loss2d[0, 0]


# speedup vs baseline: 1.3858x; 1.2486x over previous
import functools

import jax
import jax.numpy as jnp
from jax import lax
from jax.experimental import pallas as pl
from jax.experimental.pallas import tpu as pltpu
from jax.experimental.pallas import tpu_sc as plsc

F32 = jnp.float32

_N = 10000
_E = 320000
_D = 128
_H = 128
_T = 1000
_NMOL = 256
_NB = 1000
_BE = 3200
_NWORK = 32
_EPW = _E // _NWORK
_CS = 80
_NCH = _EPW // _CS
_EPT = _E // 16
_NCH4 = _EPT // _CS


def _stage1a(pos0_ref, pos1_ref, seg_ref, t_ref, tbl_ref, stats_ref, mvec_ref):
    i = pl.program_id(0)
    t = t_ref[...]
    oh_t = (t == lax.broadcasted_iota(jnp.int32, (_NB, _T), 1)).astype(F32)
    mm = jnp.dot(oh_t, tbl_ref[...], preferred_element_type=F32)
    m0 = mm[:, 0:1]
    m1 = mm[:, 1:2]
    mvec_ref[...] = jnp.concatenate([mm, jnp.zeros((_NB, 1), F32)], axis=1)
    q = m0 * pos0_ref[...] + m1 * pos1_ref[...]
    seg = seg_ref[...]
    S = (seg == lax.broadcasted_iota(jnp.int32, (_NB, _NMOL), 1)).astype(F32)
    vals = jnp.concatenate(
        [pos1_ref[...], q, m1, jnp.ones((_NB, 1), F32)], axis=1)
    part = lax.dot_general(S, vals, (((0,), (0,)), ((), ())),
                           preferred_element_type=F32)

    @pl.when(i == 0)
    def _():
        stats_ref[...] = jnp.zeros_like(stats_ref)

    stats_ref[...] += part


def _stage1b(pos0_ref, pos1_ref, h_ref, seg_ref, mvec_ref, stats_ref,
             wa_ref, wb_ref, a_ref, b_ref, xt4_ref, label_ref):
    stats = stats_ref[...]
    cnt = jnp.maximum(stats[:, 7:8], 1.0)
    mean1 = stats[:, 0:3] / cnt
    meanxt = (stats[:, 3:6] - stats[:, 6:7] * mean1) / cnt
    meanmat = jnp.concatenate([mean1, meanxt], axis=1)
    seg = seg_ref[...]
    S = (seg == lax.broadcasted_iota(jnp.int32, (_NB, _NMOL), 1)).astype(F32)
    pm = jnp.dot(S, meanmat, preferred_element_type=F32)
    mv = mvec_ref[...]
    m0 = mv[:, 0:1]
    m1 = mv[:, 1:2]
    std = mv[:, 2:3]
    pos0 = pos0_ref[...]
    q = m0 * pos0 + m1 * pos1_ref[...]
    xt = q - m1 * pm[:, 0:3] - pm[:, 3:6]
    lbl = (xt - pos0) / std
    zpad = jnp.zeros((_NB, 1), F32)
    xt4_ref[...] = jnp.concatenate([xt, zpad], axis=1)
    label_ref[...] = jnp.concatenate([lbl, zpad], axis=1)
    hv = h_ref[...]
    a_ref[...] = jnp.dot(hv, wa_ref[...], preferred_element_type=F32)
    b_ref[...] = jnp.dot(hv, wb_ref[...], preferred_element_type=F32)


def _stage3(ga_ref, gb_ref, relr_ref, wl_ref, bm1_ref, wm2_ref, bm2_ref,
            wx_ref, o1_ref, ox_ref):
    pre = ga_ref[...] + gb_ref[...]
    relt = relr_ref[...]
    sel3 = (lax.broadcasted_iota(jnp.int32, (8, 1), 0) < 3).astype(F32)
    d2 = lax.dot_general(relt * relt, sel3, (((0,), (0,)), ((), ())),
                         preferred_element_type=F32)
    z1 = pre + d2 * wl_ref[...] + bm1_ref[...]
    m1 = z1 * jax.nn.sigmoid(z1)
    z2 = jnp.dot(m1, wm2_ref[...], preferred_element_type=F32) + bm2_ref[...]
    m = z2 * jax.nn.sigmoid(z2)
    cwt = jnp.tanh(lax.dot_general(wx_ref[...], m, (((1,), (1,)), ((), ())),
                                   preferred_element_type=F32))
    o1_ref[...] = m
    ox_ref[...] = relt * cwt


def _stage5(agm0_ref, agm1_ref, agx0_ref, agx1_ref, h_ref, label_ref,
            wht_ref, whb_ref, bh_ref, out_ref):
    aggm = agm0_ref[...] + agm1_ref[...]
    aggx = (agx0_ref[...] + agx1_ref[...])[:, 0:3]
    hv = h_ref[...]
    z = (jnp.dot(hv, wht_ref[...], preferred_element_type=F32)
         + jnp.dot(aggm, whb_ref[...], preferred_element_type=F32)
         + bh_ref[...])
    ho = hv + z * jax.nn.sigmoid(z)
    diff = aggx - label_ref[...][:, 0:3]
    loss = (jnp.sum(diff * diff) / (_N * 3)
            + 1e-4 * jnp.sum(ho * ho) / (_N * _D))
    out_ref[...] = loss.reshape(1, 1)


def _sc_gather(dst_hbm, src_hbm, a_hbm, b_hbm, xt4_hbm,
               ga_hbm, gb_hbm, relr_hbm,
               xt4v, idxd, idxs, bufa, bufb, bufr, sema, semb,
               semi1, semi2, semw1, semw2, semw3):
    c = lax.axis_index("c")
    s = lax.axis_index("s")
    wid = s * 2 + c
    base = wid * _EPW
    pltpu.sync_copy(xt4_hbm, xt4v)
    lane = lax.iota(jnp.int32, 16)
    col = jnp.minimum(lane, 3)
    mask4 = lane < 4
    z16 = jnp.zeros((16,), F32)
    for r in range(8):
        for j in range(_CS // 16):
            bufr[r, pl.ds(j * 16, 16)] = z16

    def body(k, carry):
        off = base + k * _CS
        ci = pltpu.async_copy(dst_hbm.at[pl.ds(off, _CS)], idxd, semi1)
        cj = pltpu.async_copy(src_hbm.at[pl.ds(off, _CS)], idxs, semi2)
        ci.wait()
        cj.wait()
        da = pltpu.async_copy(a_hbm.at[idxd], bufa, sema)
        db = pltpu.async_copy(b_hbm.at[idxs], bufb, semb)

        def ebody(r, cc):
            for g in range(8):
                e = r * 8 + g
                ev = jnp.zeros((16,), jnp.int32) + e
                dn = plsc.load_gather(idxd, [ev])
                sn = plsc.load_gather(idxs, [ev])
                xd = plsc.load_gather(xt4v, [dn, col])
                xs = plsc.load_gather(xt4v, [sn, col])
                plsc.store_scatter(bufr, [col, ev], xd - xs, mask=mask4)
            return cc

        lax.fori_loop(0, _CS // 8, ebody, 0)
        da.wait()
        db.wait()
        w1 = pltpu.async_copy(bufa, ga_hbm.at[pl.ds(off, _CS)], semw1)
        w2 = pltpu.async_copy(bufb, gb_hbm.at[pl.ds(off, _CS)], semw2)
        w3 = pltpu.async_copy(bufr, relr_hbm.at[:, pl.ds(off, _CS)], semw3)
        w1.wait()
        w2.wait()
        w3.wait()
        return carry

    lax.fori_loop(0, _NCH, body, 0)


def _sc_scatter(o1_hbm, ox_hbm, dst_hbm, zm_hbm, zx_hbm,
                agm0_hbm, agm1_hbm, agx0_hbm, agx1_hbm,
                accm, accx, idxv, bufm, bufxt, bufx,
                sem1, sem2, sem3, sem4, sem5):
    c = lax.axis_index("c")
    s = lax.axis_index("s")
    lane = lax.iota(jnp.int32, 16)
    col = jnp.minimum(lane, 3)
    rpw = _N // 16
    pltpu.sync_copy(zm_hbm.at[pl.ds(s * rpw, rpw)],
                    accm.at[pl.ds(s * rpw, rpw)])
    pltpu.sync_copy(zx_hbm.at[pl.ds(s * rpw, rpw)],
                    accx.at[pl.ds(s * rpw, rpw)])
    plsc.subcore_barrier()
    base = (c * 16 + s) * _EPW

    def body(k, carry):
        off = base + k * _CS
        c1 = pltpu.async_copy(dst_hbm.at[pl.ds(off, _CS)], idxv, sem1)
        c2 = pltpu.async_copy(o1_hbm.at[pl.ds(off, _CS)], bufm, sem2)
        c3 = pltpu.async_copy(ox_hbm.at[:, pl.ds(off, _CS)], bufxt, sem3)
        c3.wait()

        def ebody(r, cc):
            for g in range(8):
                e = r * 8 + g
                ev = jnp.zeros((16,), jnp.int32) + e
                bufx[e, pl.ds(0, 16)] = plsc.load_gather(bufxt, [col, ev])
            return cc

        lax.fori_loop(0, _CS // 8, ebody, 0)
        c1.wait()
        c2.wait()
        a1 = pltpu.async_copy(bufm, accm.at[idxv], sem4, add=True)
        a2 = pltpu.async_copy(bufx, accx.at[idxv], sem5, add=True)
        a1.wait()
        a2.wait()
        return carry

    lax.fori_loop(0, _NCH, body, 0)
    plsc.subcore_barrier()

    @pl.when(c == 0)
    def _():
        pltpu.sync_copy(accm.at[pl.ds(s * rpw, rpw)],
                        agm0_hbm.at[pl.ds(s * rpw, rpw)])
        pltpu.sync_copy(accx.at[pl.ds(s * rpw, rpw)],
                        agx0_hbm.at[pl.ds(s * rpw, rpw)])

    @pl.when(c == 1)
    def _():
        pltpu.sync_copy(accm.at[pl.ds(s * rpw, rpw)],
                        agm1_hbm.at[pl.ds(s * rpw, rpw)])
        pltpu.sync_copy(accx.at[pl.ds(s * rpw, rpw)],
                        agx1_hbm.at[pl.ds(s * rpw, rpw)])


def kernel(pos0, pos1, h, Wm1, bm1, Wm2, bm2, Wx, Wh, bh, edge_index,
           seg_ids, t_step):
    betas = jnp.linspace(1e-5, 1e-2, _T, dtype=F32)
    var_fwd = jnp.cumsum(betas)
    var_bwd = jnp.flip(jnp.cumsum(jnp.flip(betas)))
    std_fwd = jnp.sqrt(var_fwd)
    denom = var_fwd + var_bwd
    tbl = jnp.stack([var_bwd / denom, var_fwd / denom, std_fwd], axis=1)

    seg2d = seg_ids.reshape(_N, 1)
    t2d = t_step.reshape(_N, 1)
    srci = edge_index[0]
    dsti = edge_index[1]
    nblk = _N // _NB

    stats, mvec = pl.pallas_call(
        _stage1a,
        grid=(nblk,),
        in_specs=[
            pl.BlockSpec((_NB, 3), lambda i: (i, 0)),
            pl.BlockSpec((_NB, 3), lambda i: (i, 0)),
            pl.BlockSpec((_NB, 1), lambda i: (i, 0)),
            pl.BlockSpec((_NB, 1), lambda i: (i, 0)),
            pl.BlockSpec((_T, 3), lambda i: (0, 0)),
        ],
        out_specs=[
            pl.BlockSpec((_NMOL, 8), lambda i: (0, 0)),
            pl.BlockSpec((_NB, 4), lambda i: (i, 0)),
        ],
        out_shape=[
            jax.ShapeDtypeStruct((_NMOL, 8), F32),
            jax.ShapeDtypeStruct((_N, 4), F32),
        ],
    )(pos0, pos1, seg2d, t2d, tbl)

    A, B, xt4, label4 = pl.pallas_call(
        _stage1b,
        grid=(nblk,),
        in_specs=[
            pl.BlockSpec((_NB, 3), lambda i: (i, 0)),
            pl.BlockSpec((_NB, 3), lambda i: (i, 0)),
            pl.BlockSpec((_NB, _D), lambda i: (i, 0)),
            pl.BlockSpec((_NB, 1), lambda i: (i, 0)),
            pl.BlockSpec((_NB, 4), lambda i: (i, 0)),
            pl.BlockSpec((_NMOL, 8), lambda i: (0, 0)),
            pl.BlockSpec((_D, _H), lambda i: (0, 0)),
            pl.BlockSpec((_D, _H), lambda i: (0, 0)),
        ],
        out_specs=[
            pl.BlockSpec((_NB, _H), lambda i: (i, 0)),
            pl.BlockSpec((_NB, _H), lambda i: (i, 0)),
            pl.BlockSpec((_NB, 4), lambda i: (i, 0)),
            pl.BlockSpec((_NB, 4), lambda i: (i, 0)),
        ],
        out_shape=[
            jax.ShapeDtypeStruct((_N, _H), F32),
            jax.ShapeDtypeStruct((_N, _H), F32),
            jax.ShapeDtypeStruct((_N, 4), F32),
            jax.ShapeDtypeStruct((_N, 4), F32),
        ],
    )(pos0, pos1, h, seg2d, mvec, stats, Wm1[:_D], Wm1[_D:2 * _D])

    mesh = plsc.VectorSubcoreMesh(core_axis_name="c", subcore_axis_name="s")
    gather = functools.partial(
        pl.kernel,
        out_type=[
            jax.ShapeDtypeStruct((_E, _H), F32),
            jax.ShapeDtypeStruct((_E, _H), F32),
            jax.ShapeDtypeStruct((8, _E), F32),
        ],
        mesh=mesh,
        compiler_params=pltpu.CompilerParams(use_tc_tiling_on_sc=False, needs_layout_passes=False),
        scratch_types=[
            pltpu.VMEM((_N, 4), F32),
            pltpu.VMEM((_CS,), jnp.int32),
            pltpu.VMEM((_CS,), jnp.int32),
            pltpu.VMEM((_CS, _H), F32),
            pltpu.VMEM((_CS, _H), F32),
            pltpu.VMEM((8, _CS), F32),
            pltpu.SemaphoreType.DMA,
            pltpu.SemaphoreType.DMA,
            pltpu.SemaphoreType.DMA,
            pltpu.SemaphoreType.DMA,
            pltpu.SemaphoreType.DMA,
            pltpu.SemaphoreType.DMA,
            pltpu.SemaphoreType.DMA,
        ],
    )(_sc_gather)
    GA, GB, RELR = gather(dsti, srci, A, B, xt4)

    geb = _E // _BE
    O1, OX = pl.pallas_call(
        _stage3,
        grid=(geb,),
        in_specs=[
            pl.BlockSpec((_BE, _H), lambda i: (i, 0)),
            pl.BlockSpec((_BE, _H), lambda i: (i, 0)),
            pl.BlockSpec((8, _BE), lambda i: (0, i)),
            pl.BlockSpec((1, _H), lambda i: (0, 0)),
            pl.BlockSpec((1, _H), lambda i: (0, 0)),
            pl.BlockSpec((_H, _H), lambda i: (0, 0)),
            pl.BlockSpec((1, _H), lambda i: (0, 0)),
            pl.BlockSpec((1, _H), lambda i: (0, 0)),
        ],
        out_specs=[
            pl.BlockSpec((_BE, _H), lambda i: (i, 0)),
            pl.BlockSpec((8, _BE), lambda i: (0, i)),
        ],
        out_shape=[
            jax.ShapeDtypeStruct((_E, _H), F32),
            jax.ShapeDtypeStruct((8, _E), F32),
        ],
    )(GA, GB, RELR, Wm1[2 * _D].reshape(1, _H), bm1.reshape(1, _H), Wm2,
      bm2.reshape(1, _H), Wx.reshape(1, _H))

    mesh2 = plsc.VectorSubcoreMesh(core_axis_name="c", subcore_axis_name="s")
    scatter = functools.partial(
        pl.kernel,
        out_type=[
            jax.ShapeDtypeStruct((_N, _H), F32),
            jax.ShapeDtypeStruct((_N, _H), F32),
            jax.ShapeDtypeStruct((_N, 16), F32),
            jax.ShapeDtypeStruct((_N, 16), F32),
        ],
        mesh=mesh2,
        compiler_params=pltpu.CompilerParams(use_tc_tiling_on_sc=False, needs_layout_passes=False),
        scratch_types=[
            pltpu.VMEM_SHARED((_N, _H), F32),
            pltpu.VMEM_SHARED((_N, 16), F32),
            pltpu.VMEM((_CS,), jnp.int32),
            pltpu.VMEM((_CS, _H), F32),
            pltpu.VMEM((8, _CS), F32),
            pltpu.VMEM((_CS, 16), F32),
            pltpu.SemaphoreType.DMA,
            pltpu.SemaphoreType.DMA,
            pltpu.SemaphoreType.DMA,
            pltpu.SemaphoreType.DMA,
            pltpu.SemaphoreType.DMA,
        ],
    )(_sc_scatter)
    AGM0, AGM1, AGX0, AGX1 = scatter(O1, OX, dsti, jnp.zeros((_N, _H), F32),
                                     jnp.zeros((_N, 16), F32))

    loss2d = pl.pallas_call(
        _stage5,
        grid=(1,),
        in_specs=[
            pl.BlockSpec((_N, _H), lambda i: (0, 0)),
            pl.BlockSpec((_N, _H), lambda i: (0, 0)),
            pl.BlockSpec((_N, 16), lambda i: (0, 0)),
            pl.BlockSpec((_N, 16), lambda i: (0, 0)),
            pl.BlockSpec((_N, _D), lambda i: (0, 0)),
            pl.BlockSpec((_N, 4), lambda i: (0, 0)),
            pl.BlockSpec((_D, _D), lambda i: (0, 0)),
            pl.BlockSpec((_D, _D), lambda i: (0, 0)),
            pl.BlockSpec((1, _D), lambda i: (0, 0)),
        ],
        out_specs=pl.BlockSpec((1, 1), lambda i: (0, 0)),
        out_shape=jax.ShapeDtypeStruct((1, 1), F32),
    )(AGM0, AGM1, AGX0, AGX1, h, label4, Wh[:_D], Wh[_D:], bh.reshape(1, _D))

    return loss2d[0, 0]
